# baseline (speedup 1.0000x reference)
import jax
import jax.numpy as jnp
from jax import lax
from jax.experimental import pallas as pl
from jax.experimental.pallas import tpu as pltpu
from jax.experimental.pallas import tpu_sc as plsc

N, K, D, O = 10000, 32, 128, 128

NC, NS = 2, 16
NW = NC * NS
NPAD = 10240
RPW = NPAD // NW
CH = 4
NCHUNK = RPW // CH

MM_BM = 200


def _rne_bf16_bits(x):
    u = jax.lax.bitcast_convert_type(x, jnp.int32)
    rnd = jnp.int32(0x7FFF) + (lax.shift_right_logical(u, 16) & 1)
    return lax.shift_right_logical(u + rnd, 16)


def _mm_body(h_ref, wt_ref, y_ref):
    yf = jnp.dot(h_ref[...], wt_ref[...],
                 preferred_element_type=jnp.float32)
    half = K * O // 2
    lo = _rne_bf16_bits(yf[:, :half])
    hi = _rne_bf16_bits(yf[:, half:])
    y_ref[...] = lax.shift_left(hi, 16) | (lo & jnp.int32(0xFFFF))


def _matmul(h, wt):
    return pl.pallas_call(
        _mm_body,
        grid=(N // MM_BM,),
        in_specs=[
            pl.BlockSpec((MM_BM, D), lambda i: (i, 0)),
            pl.BlockSpec((D, K * O), lambda i: (0, 0)),
        ],
        out_specs=pl.BlockSpec((MM_BM, K * O // 2), lambda i: (i, 0)),
        out_shape=jax.ShapeDtypeStruct((N, K * O // 2), jnp.int32),
    )(h, wt)


def _newton_rsqrt(sq):
    bits = plsc.bitcast(sq, jnp.int32)
    seed = jnp.int32(0x5F3759DF) - lax.shift_right_logical(bits, 1)
    y = plsc.bitcast(seed, jnp.float32)
    for _ in range(3):
        y = y * (jnp.float32(1.5) - jnp.float32(0.5) * sq * y * y)
    return jnp.where(sq == jnp.float32(0.0), jnp.float32(2.0), y)


def _sc_reduce(y2, posp, idxp):
    mesh = plsc.VectorSubcoreMesh(core_axis_name="c", subcore_axis_name="s")
    HK = CH * K // 2

    scratch = [
        pltpu.VMEM((NPAD + 16,), jnp.float32),
        pltpu.VMEM((NPAD + 16,), jnp.float32),
        pltpu.VMEM((NPAD + 16,), jnp.float32),
        pltpu.VMEM((RPW * K,), jnp.int32),
    ] + [pltpu.VMEM((CH * K,), jnp.int32) for _ in range(4)] \
      + [pltpu.VMEM((CH * K, O), jnp.int32) for _ in range(4)] \
      + [pltpu.VMEM((CH * K,), jnp.float32) for _ in range(4)] \
      + [pltpu.VMEM((CH * O,), jnp.float32) for _ in range(4)] \
      + [pltpu.SemaphoreType.DMA for _ in range(8)]

    def body(y_hbm, pos_hbm, idx_hbm, out_hbm, *bufs):
        posx, posy, posz, idxv = bufs[0:4]
        gidx = bufs[4:8]
        rows = bufs[8:12]
        inv = bufs[12:16]
        outb = bufs[16:20]
        sem = bufs[20:24]
        semo = bufs[24:28]
        wid = lax.axis_index("s") * NC + lax.axis_index("c")
        base = wid * RPW
        pltpu.sync_copy(pos_hbm.at[pl.ds(0, NPAD)], posx.at[pl.ds(0, NPAD)])
        pltpu.sync_copy(pos_hbm.at[pl.ds(NPAD, NPAD)],
                        posy.at[pl.ds(0, NPAD)])
        pltpu.sync_copy(pos_hbm.at[pl.ds(2 * NPAD, NPAD)],
                        posz.at[pl.ds(0, NPAD)])
        pltpu.sync_copy(idx_hbm.at[pl.ds(base * K, RPW * K)], idxv)
        lane = lax.iota(jnp.int32, 16)

        nch = jnp.where(base + RPW <= N, NCHUNK, (N - base) // CH)

        def prepare(c, p):
            for ii in range(CH):
                r = c * CH + ii
                g = base + r
                xi = posx[pl.ds(g, 16)][0]
                yi = posy[pl.ds(g, 16)][0]
                zi = posz[pl.ds(g, 16)][0]
                for half in range(2):
                    v = idxv[pl.ds(r * K + half * 16, 16)]
                    gidx[p][pl.ds(ii * K + half * 16, 16)] = (
                        v * (K // 2) + lane)
                    dx = xi - plsc.load_gather(posx, [v])
                    dy = yi - plsc.load_gather(posy, [v])
                    dz = zi - plsc.load_gather(posz, [v])
                    sq = dx * dx + dy * dy + dz * dz
                    inv[p][pl.ds(ii * K + half * 16, 16)] = _newton_rsqrt(sq)

        def start(p):
            pltpu.async_copy(y_hbm.at[gidx[p]], rows[p], sem[p])

        def wait(p):
            pltpu.make_async_copy(y_hbm.at[gidx[p]], rows[p], sem[p]).wait()

        def consume(c, p, first):
            @pl.when(jnp.logical_not(first))
            def _():
                pltpu.make_async_copy(
                    outb[p], out_hbm.at[pl.ds(0, CH * O)], semo[p]).wait()

            def row_body(ii, carry):
                iv0 = inv[p][pl.ds(ii * K, 16)]
                iv1 = inv[p][pl.ds(ii * K + 16, 16)]
                acc = [jnp.zeros((16,), jnp.float32) for _ in range(8)]
                for kk in range(K):
                    s = (iv0 if kk < 16 else iv1)[kk % 16]
                    j = ii * K + kk
                    for m in range(8):
                        pk = plsc.bitcast(
                            rows[p][j, pl.ds(m * 16, 16)], jnp.bfloat16)
                        a, b = plsc.unpack(
                            pk, format=plsc.PackFormat.INTERLEAVED)
                        acc[m] = acc[m] + s * (a if kk < 16 else b)
                for u in range(8):
                    outb[p][pl.ds(ii * O + u * 16, 16)] = acc[u]
                return carry

            lax.fori_loop(0, CH, row_body, 0)
            pltpu.async_copy(
                outb[p], out_hbm.at[pl.ds((base + c * CH) * O, CH * O)],
                semo[p])

        for p in range(3):
            prepare(p, p)
            start(p)

        def step(t, carry):
            for p in range(4):
                c = 4 * t + p
                wait(p)
                consume(c, p, t == 0)

                @pl.when(c + 3 < nch)
                def _():
                    prepare(c + 3, (p + 3) % 4)
                    start((p + 3) % 4)
            return carry

        lax.fori_loop(0, nch // 4, step, 0)
        for p in range(4):
            pltpu.make_async_copy(
                outb[p], out_hbm.at[pl.ds(0, CH * O)], semo[p]).wait()

    return pl.kernel(
        body,
        out_type=jax.ShapeDtypeStruct((N * O,), jnp.float32),
        mesh=mesh,
        scratch_types=scratch,
        compiler_params=pltpu.CompilerParams(needs_layout_passes=False),
    )(y2, posp, idxp)


@jax.jit
def kernel(h, pos, neighbor_idx, W):
    wt = W.reshape(K, D, O).transpose(1, 0, 2).reshape(D, K * O)
    y = _matmul(h.astype(jnp.bfloat16), wt.astype(jnp.bfloat16))
    y2 = y.reshape(N * K // 2, O)
    posp = (jnp.zeros((3, NPAD), jnp.float32)
            .at[:, :N].set(pos.T).reshape(3 * NPAD))
    idxp = (jnp.zeros((NPAD, K), jnp.int32)
            .at[:N].set(neighbor_idx).reshape(NPAD * K))
    out = _sc_reduce(y2, posp, idxp)
    return out.reshape(N, O)


# speedup vs baseline: 1.0678x; 1.0678x over previous
import jax
import jax.numpy as jnp
from jax import lax
from jax.experimental import pallas as pl
from jax.experimental.pallas import tpu as pltpu
from jax.experimental.pallas import tpu_sc as plsc

N, K, D, O = 10000, 32, 128, 128

NC, NS = 2, 16
NW = NC * NS
NPAD = 10240
RPW = NPAD // NW
CH = 8
NCHUNK = RPW // CH

MM_BM = 400


def _rne_bf16_bits(x):
    u = jax.lax.bitcast_convert_type(x, jnp.int32)
    rnd = jnp.int32(0x7FFF) + (lax.shift_right_logical(u, 16) & 1)
    return lax.shift_right_logical(u + rnd, 16)


def _mm_body(h_ref, wt_ref, y_ref):
    yf = jnp.dot(h_ref[...], wt_ref[...],
                 preferred_element_type=jnp.float32)
    half = K * O // 2
    lo = _rne_bf16_bits(yf[:, :half])
    hi = _rne_bf16_bits(yf[:, half:])
    y_ref[...] = lax.shift_left(hi, 16) | (lo & jnp.int32(0xFFFF))


def _matmul(h, wt):
    return pl.pallas_call(
        _mm_body,
        grid=(N // MM_BM,),
        in_specs=[
            pl.BlockSpec((MM_BM, D), lambda i: (i, 0)),
            pl.BlockSpec((D, K * O), lambda i: (0, 0)),
        ],
        out_specs=pl.BlockSpec((MM_BM, K * O // 2), lambda i: (i, 0)),
        out_shape=jax.ShapeDtypeStruct((N, K * O // 2), jnp.int32),
    )(h, wt)


def _newton_rsqrt(sq):
    bits = plsc.bitcast(sq, jnp.int32)
    seed = jnp.int32(0x5F3759DF) - lax.shift_right_logical(bits, 1)
    y = plsc.bitcast(seed, jnp.float32)
    for _ in range(3):
        y = y * (jnp.float32(1.5) - jnp.float32(0.5) * sq * y * y)
    return jnp.where(sq == jnp.float32(0.0), jnp.float32(2.0), y)


def _sc_reduce(y2, posp, idxp):
    mesh = plsc.VectorSubcoreMesh(core_axis_name="c", subcore_axis_name="s")
    HK = CH * K // 2

    scratch = [
        pltpu.VMEM((NPAD + 16,), jnp.float32),
        pltpu.VMEM((NPAD + 16,), jnp.float32),
        pltpu.VMEM((NPAD + 16,), jnp.float32),
        pltpu.VMEM((RPW * K,), jnp.int32),
        pltpu.VMEM((HK,), jnp.int32),
        pltpu.VMEM((HK,), jnp.int32),
        pltpu.VMEM((HK,), jnp.int32),
        pltpu.VMEM((HK,), jnp.int32),
        pltpu.VMEM((CH * K, O), jnp.int32),
        pltpu.VMEM((CH * K, O), jnp.int32),
        pltpu.VMEM((CH * K,), jnp.float32),
        pltpu.VMEM((CH * K,), jnp.float32),
        pltpu.VMEM((CH * O,), jnp.float32),
        pltpu.VMEM((CH * O,), jnp.float32),
        pltpu.SemaphoreType.DMA,
        pltpu.SemaphoreType.DMA,
        pltpu.SemaphoreType.DMA,
        pltpu.SemaphoreType.DMA,
    ]

    def body(y_hbm, pos_hbm, idx_hbm, out_hbm,
             posx, posy, posz, idxv, gidx0a, gidx0b, gidx1a, gidx1b,
             rows0, rows1, inv0, inv1, outb0, outb1,
             sem0, sem1, semo0, semo1):
        gidx0 = (gidx0a, gidx0b)
        gidx1 = (gidx1a, gidx1b)
        wid = lax.axis_index("s") * NC + lax.axis_index("c")
        base = wid * RPW
        pltpu.sync_copy(pos_hbm.at[pl.ds(0, NPAD)], posx.at[pl.ds(0, NPAD)])
        pltpu.sync_copy(pos_hbm.at[pl.ds(NPAD, NPAD)],
                        posy.at[pl.ds(0, NPAD)])
        pltpu.sync_copy(pos_hbm.at[pl.ds(2 * NPAD, NPAD)],
                        posz.at[pl.ds(0, NPAD)])
        pltpu.sync_copy(idx_hbm.at[pl.ds(base * K, RPW * K)], idxv)
        lane = lax.iota(jnp.int32, 16)

        def prepare(c, gidx, inv):
            for ii in range(CH):
                r = c * CH + ii
                g = base + r
                gi = gidx[ii // (CH // 2)]
                go = (ii % (CH // 2)) * K
                xi = posx[pl.ds(g, 16)][0]
                yi = posy[pl.ds(g, 16)][0]
                zi = posz[pl.ds(g, 16)][0]
                for half in range(2):
                    v = idxv[pl.ds(r * K + half * 16, 16)]
                    gi[pl.ds(go + half * 16, 16)] = v * (K // 2) + lane
                    dx = xi - plsc.load_gather(posx, [v])
                    dy = yi - plsc.load_gather(posy, [v])
                    dz = zi - plsc.load_gather(posz, [v])
                    sq = dx * dx + dy * dy + dz * dz
                    inv[pl.ds(ii * K + half * 16, 16)] = _newton_rsqrt(sq)

        def consume(c, rows, inv, outb, semo, first):
            @pl.when(jnp.logical_not(first))
            def _():
                pltpu.make_async_copy(
                    outb, out_hbm.at[pl.ds(0, CH * O)], semo).wait()

            def row_body(ii, carry):
                iv0 = inv[pl.ds(ii * K, 16)]
                iv1 = inv[pl.ds(ii * K + 16, 16)]
                acc = [jnp.zeros((16,), jnp.float32) for _ in range(8)]
                for kk in range(K):
                    s = (iv0 if kk < 16 else iv1)[kk % 16]
                    j = ii * K + kk
                    for m in range(8):
                        pk = plsc.bitcast(
                            rows[j, pl.ds(m * 16, 16)], jnp.bfloat16)
                        a, b = plsc.unpack(
                            pk, format=plsc.PackFormat.INTERLEAVED)
                        acc[m] = acc[m] + s * (a if kk < 16 else b)
                for u in range(8):
                    outb[pl.ds(ii * O + u * 16, 16)] = acc[u]
                return carry

            lax.fori_loop(0, CH, row_body, 0)
            pltpu.async_copy(
                outb, out_hbm.at[pl.ds((base + c * CH) * O, CH * O)], semo)

        def start(gidx, rows, sem):
            pltpu.async_copy(y_hbm.at[gidx[0]], rows.at[pl.ds(0, HK)], sem)
            pltpu.async_copy(y_hbm.at[gidx[1]], rows.at[pl.ds(HK, HK)], sem)

        def wait(gidx, rows, sem):
            pltpu.make_async_copy(
                y_hbm.at[gidx[0]], rows.at[pl.ds(0, HK)], sem).wait()
            pltpu.make_async_copy(
                y_hbm.at[gidx[1]], rows.at[pl.ds(HK, HK)], sem).wait()

        prepare(0, gidx0, inv0)
        start(gidx0, rows0, sem0)

        def step(t, carry):
            c0 = 2 * t
            prepare(c0 + 1, gidx1, inv1)
            start(gidx1, rows1, sem1)
            wait(gidx0, rows0, sem0)
            consume(c0, rows0, inv0, outb0, semo0, t == 0)

            @pl.when(t < NCHUNK // 2 - 1)
            def _():
                prepare(c0 + 2, gidx0, inv0)
                start(gidx0, rows0, sem0)

            wait(gidx1, rows1, sem1)
            consume(c0 + 1, rows1, inv1, outb1, semo1, t == 0)
            return carry

        nsteps = jnp.where(base + RPW <= N, NCHUNK // 2,
                           (N - base) // (2 * CH))
        lax.fori_loop(0, nsteps, step, 0)
        pltpu.make_async_copy(
            outb0, out_hbm.at[pl.ds(0, CH * O)], semo0).wait()
        pltpu.make_async_copy(
            outb1, out_hbm.at[pl.ds(0, CH * O)], semo1).wait()

    return pl.kernel(
        body,
        out_type=jax.ShapeDtypeStruct((N * O,), jnp.float32),
        mesh=mesh,
        scratch_types=scratch,
        compiler_params=pltpu.CompilerParams(needs_layout_passes=False),
    )(y2, posp, idxp)


@jax.jit
def kernel(h, pos, neighbor_idx, W):
    wt = W.reshape(K, D, O).transpose(1, 0, 2).reshape(D, K * O)
    y = _matmul(h.astype(jnp.bfloat16), wt.astype(jnp.bfloat16))
    y2 = y.reshape(N * K // 2, O)
    posp = (jnp.zeros((3, NPAD), jnp.float32)
            .at[:, :N].set(pos.T).reshape(3 * NPAD))
    idxp = (jnp.zeros((NPAD, K), jnp.int32)
            .at[:N].set(neighbor_idx).reshape(NPAD * K))
    out = _sc_reduce(y2, posp, idxp)
    return out.reshape(N, O)
